# baseline (speedup 1.0000x reference)
import functools

import jax
import jax.numpy as jnp
from jax import lax
from jax.experimental import pallas as pl
from jax.experimental.pallas import tpu as pltpu
from jax.experimental.pallas import tpu_sc as plsc

N = 16384
D = 256
K = 8192
COMMITMENT = 0.25

TN = 512

NC = 2
NS = 16
L = 16
NW = NC * NS
BPW = N // NW
CH = 128
NCH = BPW // CH


def _argmin_body(z_ref, w_ref, z2_ref, w2_ref, idx_ref):
    z = z_ref[...]
    dot = lax.dot_general(z.astype(jnp.bfloat16),
                          w_ref[...].astype(jnp.bfloat16),
                          (((1,), (1,)), ((), ())),
                          preferred_element_type=jnp.float32)
    dist = (z2_ref[...][:, None] + w2_ref[...][None, :]) - 2.0 * dot
    ids = lax.broadcasted_iota(jnp.int32, dist.shape, 1)
    acc_v = None
    acc_i = None
    for lo, hi in ((0, 2736), (2736, 5472), (5472, K)):
        dc = jnp.where((ids >= lo) & (ids < hi), dist, jnp.float32(jnp.inf))
        m = jnp.min(dc, axis=1)
        im = jnp.min(jnp.where(dc == m[:, None], ids, jnp.int32(K)), axis=1)
        mr = m.astype(jnp.bfloat16).astype(jnp.float32)
        if acc_v is None:
            acc_v, acc_i = mr, im
        else:
            keep = acc_v <= m
            acc_v = jnp.where(keep, acc_v, mr)
            acc_i = jnp.where(keep, acc_i, im)
    idx_ref[...] = acc_i


def _argmin(z, w, z2, w2):
    return pl.pallas_call(
        _argmin_body,
        grid=(N // TN,),
        in_specs=[
            pl.BlockSpec((TN, D), lambda i: (i, 0)),
            pl.BlockSpec((K, D), lambda i: (0, 0)),
            pl.BlockSpec((TN,), lambda i: (i,)),
            pl.BlockSpec((K,), lambda i: (0,)),
        ],
        out_specs=pl.BlockSpec((TN,), lambda i: (i,)),
        out_shape=jax.ShapeDtypeStruct((N,), jnp.int32),
        compiler_params=pltpu.CompilerParams(
            dimension_semantics=("arbitrary",),
        ),
    )(z, w, z2, w2)


def _sc_body(z_hbm, w_hbm, idx_hbm, zeros_hbm,
             qst_hbm, cnt_hbm, part_hbm,
             idx_v, rows_v, z_v, ones_v, acc_v, cnt_sh, sem):
    cid = lax.axis_index("c")
    sid = lax.axis_index("s")
    wid = sid * NC + cid
    base = wid * BPW

    @pl.when(sid == 0)
    def _():
        pltpu.sync_copy(zeros_hbm, cnt_sh)

    for j in range(NCH):
        pltpu.sync_copy(idx_hbm.at[pl.ds(base + j * CH, CH)], idx_v.at[j])

    def _fill(i, _):
        ones_v[pl.ds(i * L, L)] = jnp.full((L,), 1.0, jnp.float32)
        return 0
    lax.fori_loop(0, CH // L, _fill, 0)

    plsc.subcore_barrier()

    for j in range(NCH):
        pltpu.sync_copy(ones_v, cnt_sh.at[idx_v.at[j]], add=True)

    acc = jnp.zeros((L,), jnp.float32)
    for j in range(NCH):
        rowbase = base + j * CH
        cp = pltpu.async_copy(w_hbm.at[idx_v.at[j]], rows_v, sem)
        pltpu.sync_copy(z_hbm.at[pl.ds(rowbase, CH)], z_v)
        cp.wait()

        def _row(r, a):
            for c in range(D // L):
                sl = pl.ds(c * L, L)
                zv = z_v[r, sl]
                qv = rows_v[r, sl]
                d = qv - zv
                rows_v[r, sl] = zv + d
                a = a + d * d
            return a
        acc = lax.fori_loop(0, CH, _row, acc)

        pltpu.sync_copy(rows_v, qst_hbm.at[pl.ds(rowbase, CH)])

    acc_v[...] = acc
    pltpu.sync_copy(acc_v, part_hbm.at[wid])

    plsc.subcore_barrier()

    @pl.when(sid == 0)
    def _():
        pltpu.sync_copy(cnt_sh, cnt_hbm.at[cid])


def _sc_gather(z, w, idx, zeros_k):
    mesh = plsc.VectorSubcoreMesh(core_axis_name="c", subcore_axis_name="s",
                                  num_cores=NC, num_subcores=NS)
    fn = pl.kernel(
        _sc_body,
        out_type=(
            jax.ShapeDtypeStruct((N, D), jnp.float32),
            jax.ShapeDtypeStruct((NC, K), jnp.float32),
            jax.ShapeDtypeStruct((NW, L), jnp.float32),
        ),
        mesh=mesh,
        scratch_types=[
            pltpu.VMEM((NCH, CH), jnp.int32),
            pltpu.VMEM((CH, D), jnp.float32),
            pltpu.VMEM((CH, D), jnp.float32),
            pltpu.VMEM((CH,), jnp.float32),
            pltpu.VMEM((L,), jnp.float32),
            pltpu.VMEM_SHARED((K,), jnp.float32),
            pltpu.SemaphoreType.DMA,
        ],
    )
    return fn(z, w, idx, zeros_k)


def _scalars_body(cnt_ref, part_ref, loss_ref, perp_ref):
    counts = cnt_ref[0, :] + cnt_ref[1, :]
    avg = counts / jnp.float32(N)
    ent = jnp.sum(avg * jnp.log(avg + 1e-10))
    perp = jnp.minimum(jnp.exp(-ent), jnp.float32(K))
    perp = jnp.where(jnp.isnan(perp) | jnp.isinf(perp), jnp.float32(0.0),
                     perp)
    loss = COMMITMENT * (jnp.sum(part_ref[...]) / jnp.float32(N * D))
    loss_ref[0, 0] = loss
    perp_ref[0, 0] = perp


def _scalars(cnt, part):
    return pl.pallas_call(
        _scalars_body,
        out_specs=(pl.BlockSpec(memory_space=pltpu.SMEM),
                   pl.BlockSpec(memory_space=pltpu.SMEM)),
        out_shape=(jax.ShapeDtypeStruct((1, 1), jnp.float32),
                   jax.ShapeDtypeStruct((1, 1), jnp.float32)),
    )(cnt, part)


@jax.jit
def kernel(inputs, W):
    B, T, Dd = inputs.shape
    z = inputs.reshape(N, D)
    z2 = jnp.sum(inputs ** 2, axis=2).reshape(N)
    w2 = jnp.sum(W ** 2, axis=1)
    idx = _argmin(z, W, z2, w2)
    zeros_k = jnp.zeros((K,), jnp.float32)
    qst, cnt, part = _sc_gather(z, W, idx, zeros_k)
    loss, perp = _scalars(cnt, part)
    return (qst.reshape(B, T, Dd), loss[0, 0], perp[0, 0])
